# Initial kernel scaffold; baseline (speedup 1.0000x reference)
import jax
import jax.numpy as jnp
from jax.experimental import pallas as pl


def kernel(x, edge_index, subgraphs, W1, b1, W2, b2, Wc, bc):
    raise NotImplementedError("write your pallas kernel here")


# speedup vs baseline: 8.1846x; 8.1846x over previous
import functools

import jax
import jax.numpy as jnp
from jax import lax
from jax.experimental import pallas as pl
from jax.experimental.pallas import tpu as pltpu
from jax.experimental.pallas import tpu_sc as plsc

N = 10000
E = 320000
D = 128
H = 128
C = 16
S = 512
L = 128

NC = 2
NS = 16
NW = NC * NS
KC = 80
TOT_E = NW * KC * 128
NTAB = 10112
KP = S * L // (NW * 128)

_MESH = plsc.VectorSubcoreMesh(core_axis_name="c", subcore_axis_name="s")


def _gs_body(k_chunks, n_acc, table, srcix, dstix, zeros, out,
             src_v, dst_v, rows_v, acc_sh, sem):
    c = lax.axis_index("c")
    s = lax.axis_index("s")
    wid = s * NC + c

    @pl.when(s == 0)
    def _zero():
        pltpu.sync_copy(zeros.at[pl.ds(0, n_acc)], acc_sh)

    pltpu.sync_copy(srcix.at[pl.ds(wid * k_chunks, k_chunks)], src_v)
    pltpu.sync_copy(dstix.at[pl.ds(wid * k_chunks, k_chunks)], dst_v)
    plsc.subcore_barrier()

    def body(j, carry):
        pltpu.async_copy(table.at[src_v.at[j]], rows_v, sem).wait()
        pltpu.sync_copy(rows_v, acc_sh.at[dst_v.at[j]], add=True)
        return carry

    lax.fori_loop(0, k_chunks, body, 0)
    plsc.subcore_barrier()
    rpt = n_acc // NS
    pltpu.sync_copy(acc_sh.at[pl.ds(s * rpt, rpt)],
                    out.at[c, pl.ds(s * rpt, rpt)])


def _gs_pass(table, srcix, dstix, zeros, k_chunks, n_acc):
    f = pl.kernel(
        functools.partial(_gs_body, k_chunks, n_acc),
        out_type=jax.ShapeDtypeStruct((NC, n_acc, H), jnp.float32),
        mesh=_MESH,
        scratch_types=[
            pltpu.VMEM((k_chunks, 128), jnp.int32),
            pltpu.VMEM((k_chunks, 128), jnp.int32),
            pltpu.VMEM((128, H), jnp.float32),
            pltpu.VMEM_SHARED((n_acc, H), jnp.float32),
            pltpu.SemaphoreType.DMA,
        ],
    )
    return f(table, srcix, dstix, zeros)


EPT = KC * 128


def _deg_body(dst1d, zeros1d, out, dst_v, hist_v):
    c = lax.axis_index("c")
    s = lax.axis_index("s")
    wid = s * NC + c

    pltpu.sync_copy(zeros1d, hist_v)
    pltpu.sync_copy(dst1d.at[pl.ds(wid * EPT, EPT)], dst_v)

    def body(j, carry):
        for i in range(8):
            idx = dst_v[pl.ds(j * 128 + i * 16, 16)]
            counts, last = plsc.scan_count(idx)
            plsc.addupdate_scatter(
                hist_v, [idx], counts.astype(jnp.float32), mask=last)
        return carry

    lax.fori_loop(0, KC, body, 0)
    pltpu.sync_copy(hist_v, out.at[wid])


def _deg_pass(dst1d, zeros1d):
    f = pl.kernel(
        _deg_body,
        out_type=jax.ShapeDtypeStruct((NW, NTAB), jnp.float32),
        mesh=_MESH,
        scratch_types=[
            pltpu.VMEM((EPT,), jnp.int32),
            pltpu.VMEM((NTAB,), jnp.float32),
        ],
        compiler_params=pltpu.CompilerParams(needs_layout_passes=False),
    )
    return f(dst1d, zeros1d)


def _dis_from(degp_ref):
    deg = jnp.sum(degp_ref[...], axis=0)[:, None] + 1.0
    rows = lax.broadcasted_iota(jnp.int32, (NTAB, 1), 0)
    return jnp.where(rows < N, lax.rsqrt(deg), 0.0)


def _tc1_body(xp_ref, w1_ref, degp_ref, ht1_ref):
    dis = _dis_from(degp_ref)
    hw = jnp.dot(xp_ref[...], w1_ref[...], preferred_element_type=jnp.float32)
    ht1_ref[...] = dis * hw


def _tc2_body(p_ref, ht1_ref, degp_ref, b1_ref, w2_ref, ht2_ref):
    dis = _dis_from(degp_ref)
    h1 = dis * (p_ref[0] + p_ref[1] + ht1_ref[...]) + b1_ref[...]
    h1 = jnp.maximum(h1, 0.0)
    ht2_ref[...] = dis * jnp.dot(h1, w2_ref[...],
                                 preferred_element_type=jnp.float32)


def _tc3_body(q_ref, ht2_ref, degp_ref, b2_ref, h2_ref):
    dis = _dis_from(degp_ref)
    h2_ref[...] = dis * (q_ref[0] + q_ref[1] + ht2_ref[...]) + b2_ref[...]


def _tc4_body(pp_ref, wc_ref, bc_ref, out_ref):
    emb = (pp_ref[0] + pp_ref[1]) * (1.0 / L)
    out_ref[...] = jnp.dot(emb, wc_ref[...],
                           preferred_element_type=jnp.float32) + bc_ref[...]


def _tc_call(body, out_shape, *args):
    return pl.pallas_call(
        body, out_shape=jax.ShapeDtypeStruct(out_shape, jnp.float32))(*args)


@jax.jit
def kernel(x, edge_index, subgraphs, W1, b1, W2, b2, Wc, bc):
    x2 = jnp.squeeze(x, axis=1)
    xp = jnp.zeros((NTAB, D), jnp.float32).at[:N].set(x2)

    src = edge_index[0].astype(jnp.int32)
    dst = edge_index[1].astype(jnp.int32)
    pad = TOT_E - E
    padv = jnp.full((pad,), N, jnp.int32)
    srcp = jnp.concatenate([src, padv]).reshape(NW * KC, 128)
    dstp = jnp.concatenate([dst, padv]).reshape(NW * KC, 128)

    sub_ix = subgraphs.astype(jnp.int32)
    pool_dst = jnp.broadcast_to(
        jnp.arange(S, dtype=jnp.int32)[:, None], (S, L))

    zeros_big = jnp.zeros((NTAB, H), jnp.float32)
    zeros1d = jnp.zeros((NTAB,), jnp.float32)

    degp = _deg_pass(dstp.reshape(-1), zeros1d)
    ht1 = _tc_call(_tc1_body, (NTAB, H), xp, W1, degp)
    p1 = _gs_pass(ht1, srcp, dstp, zeros_big, KC, NTAB)
    ht2 = _tc_call(_tc2_body, (NTAB, H), p1, ht1, degp, b1, W2)
    p2 = _gs_pass(ht2, srcp, dstp, zeros_big, KC, NTAB)
    h2 = _tc_call(_tc3_body, (NTAB, H), p2, ht2, degp, b2)
    pp = _gs_pass(h2, sub_ix, pool_dst, zeros_big, KP, S)
    out = _tc_call(_tc4_body, (S, C), pp, Wc, bc)
    return out


# speedup vs baseline: 8.9129x; 1.0890x over previous
import functools

import jax
import jax.numpy as jnp
from jax import lax
from jax.experimental import pallas as pl
from jax.experimental.pallas import tpu as pltpu
from jax.experimental.pallas import tpu_sc as plsc

N = 10000
E = 320000
D = 128
H = 128
C = 16
S = 512
L = 128

NC = 2
NS = 16
NW = NC * NS
KC = 80
TOT_E = NW * KC * 128
NTAB = 10112
KP = S * L // (NW * 128)

_MESH = plsc.VectorSubcoreMesh(core_axis_name="c", subcore_axis_name="s")


NBUF = 2


def _gs_body(k_chunks, n_acc, table, srcix, dstix, zeros, out,
             src_v, dst_v, rows_v, acc_sh, gsem, ssem):
    c = lax.axis_index("c")
    s = lax.axis_index("s")
    wid = s * NC + c
    g = k_chunks // 2

    @pl.when(s == 0)
    def _zero():
        pltpu.sync_copy(zeros.at[pl.ds(0, n_acc)], acc_sh)

    plsc.subcore_barrier()

    def _gather_start(j, b):
        pltpu.async_copy(table.at[src_v.at[j]], rows_v[b], gsem[b])

    def _gather_wait(j, b):
        pltpu.make_async_copy(table.at[src_v.at[j]], rows_v[b],
                              gsem[b]).wait()

    def _scatter_start(j, b):
        pltpu.async_copy(rows_v[b], acc_sh.at[dst_v.at[j]], ssem[b],
                         add=True)

    def _scatter_wait(j, b):
        pltpu.make_async_copy(rows_v[b], acc_sh.at[dst_v.at[j]],
                              ssem[b]).wait()

    for h in range(2):
        pltpu.sync_copy(srcix.at[pl.ds(wid * k_chunks + h * g, g)], src_v)
        pltpu.sync_copy(dstix.at[pl.ds(wid * k_chunks + h * g, g)], dst_v)

        for b in range(NBUF):
            _gather_start(b, b)

        def body(t, carry):
            j0 = t * NBUF
            for b in range(NBUF):
                _gather_wait(j0 + b, b)
                _scatter_start(j0 + b, b)
            for b in range(NBUF):
                _scatter_wait(j0 + b, b)
                _gather_start(j0 + NBUF + b, b)
            return carry

        lax.fori_loop(0, g // NBUF - 1, body, 0)
        j0 = g - NBUF
        for b in range(NBUF):
            _gather_wait(j0 + b, b)
            _scatter_start(j0 + b, b)
        for b in range(NBUF):
            _scatter_wait(j0 + b, b)

    plsc.subcore_barrier()
    rpt = n_acc // NS
    pltpu.sync_copy(acc_sh.at[pl.ds(s * rpt, rpt)],
                    out.at[c, pl.ds(s * rpt, rpt)])


def _gs_pass(table, srcix, dstix, zeros, k_chunks, n_acc):
    f = pl.kernel(
        functools.partial(_gs_body, k_chunks, n_acc),
        out_type=jax.ShapeDtypeStruct((NC, n_acc, H), jnp.float32),
        mesh=_MESH,
        scratch_types=[
            pltpu.VMEM((k_chunks // 2, 128), jnp.int32),
            pltpu.VMEM((k_chunks // 2, 128), jnp.int32),
            [pltpu.VMEM((128, H), jnp.float32) for _ in range(NBUF)],
            pltpu.VMEM_SHARED((n_acc, H), jnp.float32),
            [pltpu.SemaphoreType.DMA for _ in range(NBUF)],
            [pltpu.SemaphoreType.DMA for _ in range(NBUF)],
        ],
    )
    return f(table, srcix, dstix, zeros)


EPT = KC * 128


def _deg_body(dst1d, zeros1d, out, dst_v, hist_v):
    c = lax.axis_index("c")
    s = lax.axis_index("s")
    wid = s * NC + c

    pltpu.sync_copy(zeros1d, hist_v)
    pltpu.sync_copy(dst1d.at[pl.ds(wid * EPT, EPT)], dst_v)

    def body(j, carry):
        for i in range(8):
            idx = dst_v[pl.ds(j * 128 + i * 16, 16)]
            counts, last = plsc.scan_count(idx)
            plsc.addupdate_scatter(
                hist_v, [idx], counts.astype(jnp.float32), mask=last)
        return carry

    lax.fori_loop(0, KC, body, 0)
    pltpu.sync_copy(hist_v, out.at[wid])


def _deg_pass(dst1d, zeros1d):
    f = pl.kernel(
        _deg_body,
        out_type=jax.ShapeDtypeStruct((NW, NTAB), jnp.float32),
        mesh=_MESH,
        scratch_types=[
            pltpu.VMEM((EPT,), jnp.int32),
            pltpu.VMEM((NTAB,), jnp.float32),
        ],
        compiler_params=pltpu.CompilerParams(needs_layout_passes=False),
    )
    return f(dst1d, zeros1d)


def _dis_from(degp_ref):
    deg = jnp.sum(degp_ref[...], axis=0)[:, None] + 1.0
    rows = lax.broadcasted_iota(jnp.int32, (NTAB, 1), 0)
    return jnp.where(rows < N, lax.rsqrt(deg), 0.0)


def _tc1_body(xp_ref, w1_ref, degp_ref, ht1_ref):
    dis = _dis_from(degp_ref)
    hw = jnp.dot(xp_ref[...], w1_ref[...], preferred_element_type=jnp.float32)
    ht1_ref[...] = dis * hw


def _tc2_body(p_ref, ht1_ref, degp_ref, b1_ref, w2_ref, ht2_ref):
    dis = _dis_from(degp_ref)
    h1 = dis * (p_ref[0] + p_ref[1] + ht1_ref[...]) + b1_ref[...]
    h1 = jnp.maximum(h1, 0.0)
    ht2_ref[...] = dis * jnp.dot(h1, w2_ref[...],
                                 preferred_element_type=jnp.float32)


def _tc3_body(q_ref, ht2_ref, degp_ref, b2_ref, h2_ref):
    dis = _dis_from(degp_ref)
    h2_ref[...] = dis * (q_ref[0] + q_ref[1] + ht2_ref[...]) + b2_ref[...]


def _tc4_body(pp_ref, wc_ref, bc_ref, out_ref):
    emb = (pp_ref[0] + pp_ref[1]) * (1.0 / L)
    out_ref[...] = jnp.dot(emb, wc_ref[...],
                           preferred_element_type=jnp.float32) + bc_ref[...]


def _tc_call(body, out_shape, *args):
    return pl.pallas_call(
        body, out_shape=jax.ShapeDtypeStruct(out_shape, jnp.float32))(*args)


@jax.jit
def kernel(x, edge_index, subgraphs, W1, b1, W2, b2, Wc, bc):
    x2 = jnp.squeeze(x, axis=1)
    xp = jnp.zeros((NTAB, D), jnp.float32).at[:N].set(x2)

    src = edge_index[0].astype(jnp.int32)
    dst = edge_index[1].astype(jnp.int32)
    pad = TOT_E - E
    padv = jnp.full((pad,), N, jnp.int32)
    srcp = jnp.concatenate([src, padv]).reshape(NW * KC, 128)
    dstp = jnp.concatenate([dst, padv]).reshape(NW * KC, 128)

    sub_ix = subgraphs.astype(jnp.int32)
    pool_dst = jnp.broadcast_to(
        jnp.arange(S, dtype=jnp.int32)[:, None], (S, L))

    zeros_big = jnp.zeros((NTAB, H), jnp.float32)
    zeros1d = jnp.zeros((NTAB,), jnp.float32)

    degp = _deg_pass(dstp.reshape(-1), zeros1d)
    ht1 = _tc_call(_tc1_body, (NTAB, H), xp, W1, degp)
    p1 = _gs_pass(ht1, srcp, dstp, zeros_big, KC, NTAB)
    ht2 = _tc_call(_tc2_body, (NTAB, H), p1, ht1, degp, b1, W2)
    p2 = _gs_pass(ht2, srcp, dstp, zeros_big, KC, NTAB)
    h2 = _tc_call(_tc3_body, (NTAB, H), p2, ht2, degp, b2)
    pp = _gs_pass(h2, sub_ix, pool_dst, zeros_big, KP, S)
    out = _tc_call(_tc4_body, (S, C), pp, Wc, bc)
    return out


# speedup vs baseline: 24.3655x; 2.7337x over previous
import functools

import jax
import jax.numpy as jnp
from jax import lax
from jax.experimental import pallas as pl
from jax.experimental.pallas import tpu as pltpu
from jax.experimental.pallas import tpu_sc as plsc

N = 10000
E = 320000
D = 128
H = 128
C = 16
S = 512
L = 128

NC = 2
NS = 16
NW = NC * NS
KC = 80
TOT_E = NW * KC * 128
NTAB = 10112
KP = S * L // (NW * 128)

_MESH = plsc.VectorSubcoreMesh(core_axis_name="c", subcore_axis_name="s")


NBUF = 2


def _gs_body(k_chunks, n_acc, table, srcix, dstix, zeros, out,
             src_v, dst_v, rows_v, acc_sh, gsem, ssem):
    c = lax.axis_index("c")
    s = lax.axis_index("s")
    wid = s * NC + c
    g = k_chunks // 2

    @pl.when(s == 0)
    def _zero():
        pltpu.sync_copy(zeros.at[pl.ds(0, n_acc)], acc_sh)

    plsc.subcore_barrier()

    def _gather_start(j, b):
        pltpu.async_copy(table.at[src_v.at[j]], rows_v[b], gsem[b])

    def _gather_wait(j, b):
        pltpu.make_async_copy(table.at[src_v.at[j]], rows_v[b],
                              gsem[b]).wait()

    def _scatter_start(j, b):
        pltpu.async_copy(rows_v[b], acc_sh.at[dst_v.at[j]], ssem[b],
                         add=True)

    def _scatter_wait(j, b):
        pltpu.make_async_copy(rows_v[b], acc_sh.at[dst_v.at[j]],
                              ssem[b]).wait()

    for h in range(2):
        pltpu.sync_copy(srcix.at[pl.ds(wid * k_chunks + h * g, g)], src_v)
        pltpu.sync_copy(dstix.at[pl.ds(wid * k_chunks + h * g, g)], dst_v)

        for b in range(NBUF):
            _gather_start(b, b)

        def body(t, carry):
            j0 = t * NBUF
            for b in range(NBUF):
                _gather_wait(j0 + b, b)
                _scatter_start(j0 + b, b)
            for b in range(NBUF):
                _scatter_wait(j0 + b, b)
                _gather_start(j0 + NBUF + b, b)
            return carry

        lax.fori_loop(0, g // NBUF - 1, body, 0)
        j0 = g - NBUF
        for b in range(NBUF):
            _gather_wait(j0 + b, b)
            _scatter_start(j0 + b, b)
        for b in range(NBUF):
            _scatter_wait(j0 + b, b)

    plsc.subcore_barrier()
    rpt = n_acc // NS
    pltpu.sync_copy(acc_sh.at[pl.ds(s * rpt, rpt)],
                    out.at[c, pl.ds(s * rpt, rpt)])


def _gs_pass(table, srcix, dstix, zeros, k_chunks, n_acc):
    f = pl.kernel(
        functools.partial(_gs_body, k_chunks, n_acc),
        out_type=jax.ShapeDtypeStruct((NC, n_acc, H), jnp.float32),
        mesh=_MESH,
        scratch_types=[
            pltpu.VMEM((k_chunks // 2, 128), jnp.int32),
            pltpu.VMEM((k_chunks // 2, 128), jnp.int32),
            [pltpu.VMEM((128, H), jnp.float32) for _ in range(NBUF)],
            pltpu.VMEM_SHARED((n_acc, H), jnp.float32),
            [pltpu.SemaphoreType.DMA for _ in range(NBUF)],
            [pltpu.SemaphoreType.DMA for _ in range(NBUF)],
        ],
    )
    return f(table, srcix, dstix, zeros)


EPT = KC * 128


def _deg_body(dst1d, zeros1d, out, dst_v, hist_v):
    c = lax.axis_index("c")
    s = lax.axis_index("s")
    wid = s * NC + c

    pltpu.sync_copy(zeros1d, hist_v)
    pltpu.sync_copy(dst1d.at[pl.ds(wid * EPT, EPT)], dst_v)

    def body(j, carry):
        for i in range(8):
            idx = dst_v[pl.ds(j * 128 + i * 16, 16)]
            counts, last = plsc.scan_count(idx)
            plsc.addupdate_scatter(
                hist_v, [idx], counts.astype(jnp.float32), mask=last)
        return carry

    lax.fori_loop(0, KC, body, 0)
    pltpu.sync_copy(hist_v, out.at[wid])


def _deg_pass(dst1d, zeros1d):
    f = pl.kernel(
        _deg_body,
        out_type=jax.ShapeDtypeStruct((NW, NTAB), jnp.float32),
        mesh=_MESH,
        scratch_types=[
            pltpu.VMEM((EPT,), jnp.int32),
            pltpu.VMEM((NTAB,), jnp.float32),
        ],
        compiler_params=pltpu.CompilerParams(needs_layout_passes=False),
    )
    return f(dst1d, zeros1d)


def _dis_from(degp_ref):
    deg = jnp.sum(degp_ref[...], axis=0)[:, None] + 1.0
    rows = lax.broadcasted_iota(jnp.int32, (NTAB, 1), 0)
    return jnp.where(rows < N, lax.rsqrt(deg), 0.0)


def _tc1_body(xp_ref, w1_ref, degp_ref, ht1_ref):
    dis = _dis_from(degp_ref)
    hw = jnp.dot(xp_ref[...], w1_ref[...], preferred_element_type=jnp.float32)
    ht1_ref[...] = dis * hw


def _tc2_body(p_ref, ht1_ref, degp_ref, b1_ref, w2_ref, ht2_ref):
    dis = _dis_from(degp_ref)
    h1 = dis * (p_ref[0] + p_ref[1] + ht1_ref[...]) + b1_ref[...]
    h1 = jnp.maximum(h1, 0.0)
    ht2_ref[...] = dis * jnp.dot(h1, w2_ref[...],
                                 preferred_element_type=jnp.float32)


def _tc3_body(q_ref, ht2_ref, degp_ref, b2_ref, h2_ref):
    dis = _dis_from(degp_ref)
    h2_ref[...] = dis * (q_ref[0] + q_ref[1] + ht2_ref[...]) + b2_ref[...]


def _tc4_body(pp_ref, wc_ref, bc_ref, out_ref):
    emb = (pp_ref[0] + pp_ref[1]) * (1.0 / L)
    out_ref[...] = jnp.dot(emb, wc_ref[...],
                           preferred_element_type=jnp.float32) + bc_ref[...]


def _tc_call(body, out_shape, *args):
    return pl.pallas_call(
        body, out_shape=jax.ShapeDtypeStruct(out_shape, jnp.float32))(*args)


@jax.jit
def kernel(x, edge_index, subgraphs, W1, b1, W2, b2, Wc, bc):
    x2 = jnp.squeeze(x, axis=1)
    xp = jnp.zeros((NTAB, D), jnp.float32).at[:N].set(x2)

    src = edge_index[0].astype(jnp.int32)
    dst = edge_index[1].astype(jnp.int32)
    pad = TOT_E - E
    ar = jnp.arange(pad, dtype=jnp.int32)
    pad_src = ar % N
    pad_dst = N + ar % (NTAB - N)
    srcp = jnp.concatenate([src, pad_src]).reshape(NW * KC, 128)
    dstp = jnp.concatenate([dst, pad_dst]).reshape(NW * KC, 128)

    sub_ix = subgraphs.astype(jnp.int32)
    pool_dst = jnp.broadcast_to(
        jnp.arange(S, dtype=jnp.int32)[:, None], (S, L))

    zeros_big = jnp.zeros((NTAB, H), jnp.float32)
    zeros1d = jnp.zeros((NTAB,), jnp.float32)

    degp = _deg_pass(dstp.reshape(-1), zeros1d)
    ht1 = _tc_call(_tc1_body, (NTAB, H), xp, W1, degp)
    p1 = _gs_pass(ht1, srcp, dstp, zeros_big, KC, NTAB)
    ht2 = _tc_call(_tc2_body, (NTAB, H), p1, ht1, degp, b1, W2)
    p2 = _gs_pass(ht2, srcp, dstp, zeros_big, KC, NTAB)
    h2 = _tc_call(_tc3_body, (NTAB, H), p2, ht2, degp, b2)
    pp = _gs_pass(h2, sub_ix, pool_dst, zeros_big, KP, S)
    out = _tc_call(_tc4_body, (S, C), pp, Wc, bc)
    return out


# speedup vs baseline: 24.3683x; 1.0001x over previous
import functools

import jax
import jax.numpy as jnp
import numpy as np
from jax import lax
from jax.experimental import pallas as pl
from jax.experimental.pallas import tpu as pltpu
from jax.experimental.pallas import tpu_sc as plsc

N = 10000
E = 320000
D = 128
H = 128
C = 16
S = 512
L = 128

NC = 2
NS = 16
NW = NC * NS
KC = 80
TOT_E = NW * KC * 128
NTAB = 10112
KP = S * L // (NW * 128)

_MESH = plsc.VectorSubcoreMesh(core_axis_name="c", subcore_axis_name="s")


NBUF = 2


def _gs_body(k_chunks, n_acc, table, srcix, dstix, zeros, out,
             src_v, dst_v, rows_v, acc_sh, gsem, ssem):
    c = lax.axis_index("c")
    s = lax.axis_index("s")
    wid = s * NC + c
    g = k_chunks // 2

    zpt = n_acc // NS
    pltpu.sync_copy(zeros.at[pl.ds(s * zpt, zpt)],
                    acc_sh.at[pl.ds(s * zpt, zpt)])
    plsc.subcore_barrier()

    def _gather_start(j, b):
        pltpu.async_copy(table.at[src_v.at[j]], rows_v[b], gsem[b])

    def _gather_wait(j, b):
        pltpu.make_async_copy(table.at[src_v.at[j]], rows_v[b],
                              gsem[b]).wait()

    def _scatter_start(j, b):
        pltpu.async_copy(rows_v[b], acc_sh.at[dst_v.at[j]], ssem[b],
                         add=True)

    def _scatter_wait(j, b):
        pltpu.make_async_copy(rows_v[b], acc_sh.at[dst_v.at[j]],
                              ssem[b]).wait()

    for h in range(2):
        pltpu.sync_copy(srcix.at[pl.ds(wid * k_chunks + h * g, g)], src_v)
        pltpu.sync_copy(dstix.at[pl.ds(wid * k_chunks + h * g, g)], dst_v)

        for b in range(NBUF):
            _gather_start(b, b)

        def body(t, carry):
            j0 = t * NBUF
            for b in range(NBUF):
                _gather_wait(j0 + b, b)
                _scatter_start(j0 + b, b)
            for b in range(NBUF):
                _scatter_wait(j0 + b, b)
                _gather_start(j0 + NBUF + b, b)
            return carry

        lax.fori_loop(0, g // NBUF - 1, body, 0)
        j0 = g - NBUF
        for b in range(NBUF):
            _gather_wait(j0 + b, b)
            _scatter_start(j0 + b, b)
        for b in range(NBUF):
            _scatter_wait(j0 + b, b)

    plsc.subcore_barrier()
    rpt = n_acc // NS
    pltpu.sync_copy(acc_sh.at[pl.ds(s * rpt, rpt)],
                    out.at[c, pl.ds(s * rpt, rpt)])


def _gs_pass(table, srcix, dstix, zeros, k_chunks, n_acc):
    f = pl.kernel(
        functools.partial(_gs_body, k_chunks, n_acc),
        out_type=jax.ShapeDtypeStruct((NC, n_acc, H), jnp.float32),
        mesh=_MESH,
        scratch_types=[
            pltpu.VMEM((k_chunks // 2, 128), jnp.int32),
            pltpu.VMEM((k_chunks // 2, 128), jnp.int32),
            [pltpu.VMEM((128, H), jnp.float32) for _ in range(NBUF)],
            pltpu.VMEM_SHARED((n_acc, H), jnp.float32),
            [pltpu.SemaphoreType.DMA for _ in range(NBUF)],
            [pltpu.SemaphoreType.DMA for _ in range(NBUF)],
        ],
    )
    return f(table, srcix, dstix, zeros)


EPT = E // NW


def _deg_body(dst1d, zeros1d, out, dst_v, hist_v):
    c = lax.axis_index("c")
    s = lax.axis_index("s")
    wid = s * NC + c

    pltpu.sync_copy(zeros1d, hist_v)
    pltpu.sync_copy(dst1d.at[pl.ds(wid * EPT, EPT)], dst_v)

    def body(j, carry):
        for i in range(8):
            idx = dst_v[pl.ds(j * 128 + i * 16, 16)]
            counts, last = plsc.scan_count(idx)
            plsc.addupdate_scatter(
                hist_v, [idx], counts.astype(jnp.float32), mask=last)
        return carry

    lax.fori_loop(0, EPT // 128, body, 0)
    for i in range((EPT % 128) // 16):
        idx = dst_v[pl.ds((EPT // 128) * 128 + i * 16, 16)]
        counts, last = plsc.scan_count(idx)
        plsc.addupdate_scatter(
            hist_v, [idx], counts.astype(jnp.float32), mask=last)
    pltpu.sync_copy(hist_v, out.at[wid])


def _deg_pass(dst1d, zeros1d):
    f = pl.kernel(
        _deg_body,
        out_type=jax.ShapeDtypeStruct((NW, NTAB), jnp.float32),
        mesh=_MESH,
        scratch_types=[
            pltpu.VMEM((EPT,), jnp.int32),
            pltpu.VMEM((NTAB,), jnp.float32),
        ],
        compiler_params=pltpu.CompilerParams(needs_layout_passes=False),
    )
    return f(dst1d, zeros1d)


_PAD = TOT_E - E
_PAD_SRC = np.arange(_PAD, dtype=np.int32) % N
_PAD_DST = N + np.arange(_PAD, dtype=np.int32) % (NTAB - N)


def _dis_from(degp_ref):
    deg = jnp.sum(degp_ref[...], axis=0)[:, None] + 1.0
    rows = lax.broadcasted_iota(jnp.int32, (NTAB, 1), 0)
    return jnp.where(rows < N, lax.rsqrt(deg), 0.0)


def _tc1_body(xp_ref, w1_ref, degp_ref, ht1_ref):
    dis = _dis_from(degp_ref)
    hw = jnp.dot(xp_ref[...], w1_ref[...], preferred_element_type=jnp.float32)
    ht1_ref[...] = dis * hw


def _tc2_body(p_ref, ht1_ref, degp_ref, b1_ref, w2_ref, ht2_ref):
    dis = _dis_from(degp_ref)
    h1 = dis * (p_ref[0] + p_ref[1] + ht1_ref[...]) + b1_ref[...]
    h1 = jnp.maximum(h1, 0.0)
    ht2_ref[...] = dis * jnp.dot(h1, w2_ref[...],
                                 preferred_element_type=jnp.float32)


def _tc3_body(q_ref, ht2_ref, degp_ref, b2_ref, h2_ref):
    dis = _dis_from(degp_ref)
    h2_ref[...] = dis * (q_ref[0] + q_ref[1] + ht2_ref[...]) + b2_ref[...]


def _tc4_body(pp_ref, wc_ref, bc_ref, out_ref):
    emb = (pp_ref[0] + pp_ref[1]) * (1.0 / L)
    out_ref[...] = jnp.dot(emb, wc_ref[...],
                           preferred_element_type=jnp.float32) + bc_ref[...]


def _tc_call(body, out_shape, *args):
    return pl.pallas_call(
        body, out_shape=jax.ShapeDtypeStruct(out_shape, jnp.float32))(*args)


@jax.jit
def kernel(x, edge_index, subgraphs, W1, b1, W2, b2, Wc, bc):
    x2 = jnp.squeeze(x, axis=1)
    xp = jnp.zeros((NTAB, D), jnp.float32).at[:N].set(x2)

    src = edge_index[0].astype(jnp.int32)
    dst = edge_index[1].astype(jnp.int32)
    srcp = jnp.concatenate([src, jnp.asarray(_PAD_SRC)]).reshape(NW * KC, 128)
    dstp = jnp.concatenate([dst, jnp.asarray(_PAD_DST)]).reshape(NW * KC, 128)

    sub_ix = subgraphs.astype(jnp.int32)
    pool_dst = jnp.broadcast_to(
        jnp.arange(S, dtype=jnp.int32)[:, None], (S, L))

    zeros_big = jnp.zeros((NTAB, H), jnp.float32)
    zeros1d = jnp.zeros((NTAB,), jnp.float32)

    degp = _deg_pass(dst, zeros1d)
    ht1 = _tc_call(_tc1_body, (NTAB, H), xp, W1, degp)
    p1 = _gs_pass(ht1, srcp, dstp, zeros_big, KC, NTAB)
    ht2 = _tc_call(_tc2_body, (NTAB, H), p1, ht1, degp, b1, W2)
    p2 = _gs_pass(ht2, srcp, dstp, zeros_big, KC, NTAB)
    h2 = _tc_call(_tc3_body, (NTAB, H), p2, ht2, degp, b2)
    pp = _gs_pass(h2, sub_ix, pool_dst, zeros_big, KP, S)
    out = _tc_call(_tc4_body, (S, C), pp, Wc, bc)
    return out
